# baseline (speedup 1.0000x reference)
import numpy as np
import jax
import jax.numpy as jnp
from jax.experimental import pallas as pl
from jax.experimental.pallas import tpu as pltpu

_N = 360

_DN_T = (((1,), (1,)), ((), ()))

_WM1_SPLIT = 4
_WM2_ROWS = (104, 96)


def _fused(x_ref, w1_ref, b1_ref, w2_ref, b2_ref, w3_ref, b3_ref,
           bg_ref, bm1_ref, bm2_ref, wg_hbm, wm1_hbm, wm2_hbm,
           out_ref, wg_s, wm1_s0, wm1_s1, wm1_s2, wm1_s3,
           wm2_s0, wm2_s1, sems):
    f32 = jnp.float32

    def mm_t(v, w):
        return jax.lax.dot_general(v, w, _DN_T, preferred_element_type=f32)

    cp_g = pltpu.make_async_copy(wg_hbm, wg_s, sems.at[0])
    cp_g.start()
    wm1_bufs = (wm1_s0, wm1_s1, wm1_s2, wm1_s3)
    cp_m1 = []
    for k, buf in enumerate(wm1_bufs):
        r = 1024 // _WM1_SPLIT
        cp = pltpu.make_async_copy(wm1_hbm.at[pl.ds(k * r, r), :], buf,
                                   sems.at[1 + k])
        cp.start()
        cp_m1.append(cp)
    wm2_bufs = (wm2_s0, wm2_s1)
    cp_m2 = []
    base = 0
    for k, (r, buf) in enumerate(zip(_WM2_ROWS, wm2_bufs)):
        cp = pltpu.make_async_copy(wm2_hbm.at[pl.ds(base, r), :], buf,
                                   sems.at[1 + _WM1_SPLIT + k])
        cp.start()
        cp_m2.append(cp)
        base += r

    i = jax.lax.broadcasted_iota(jnp.int32, (_N, 1), 0)
    fi = i.astype(f32)

    ang = fi * np.float32(2.0 * np.pi / (_N - 1))
    scan = jnp.transpose(x_ref[0:1, 0:_N])
    nx = scan * jnp.cos(ang)
    ny = scan * jnp.sin(ang)

    end = (i == 0) | (i == (_N - 1))
    dis = jnp.where(end, np.float32(1.0 / np.sqrt(2.0)),
                    np.float32(1.0 / np.sqrt(3.0)))
    cd = dis * dis
    cl = jnp.where(i == 0, 0.0, dis * jnp.roll(dis, 1, axis=0))
    cu = jnp.where(i == (_N - 1), 0.0, dis * jnp.roll(dis, -1, axis=0))

    def agg(v):
        return cd * v + cl * jnp.roll(v, 1, axis=0) + cu * jnp.roll(v, -1, axis=0)

    nodes = jnp.concatenate([nx, ny], axis=1)
    xw = mm_t(nodes, w1_ref[:])
    h = jnp.maximum(agg(xw) + b1_ref[:], 0.0)

    h = jnp.maximum(agg(mm_t(h, w2_ref[:])) + b2_ref[:], 0.0)
    h = jnp.maximum(agg(mm_t(h, w3_ref[:])) + b3_ref[:], 0.0)

    g = jnp.mean(h, axis=0, keepdims=True)
    cp_g.wait()
    c = mm_t(g, wg_s[:]) + bg_ref[:]
    m_parts = []
    for cp, buf in zip(cp_m1, wm1_bufs):
        cp.wait()
        m_parts.append(mm_t(c, buf[:]))
    m = jnp.maximum(jnp.concatenate(m_parts, axis=1) + bm1_ref[:], 0.0)
    o_parts = []
    for cp, buf in zip(cp_m2, wm2_bufs):
        cp.wait()
        o_parts.append(mm_t(m, buf[:]))
    row = jnp.concatenate(o_parts, axis=1) + bm2_ref[:]
    out_ref[:] = row.reshape(1, 10, 10, 2)


@jax.jit
def _run(x, W1, b1, W2, b2, W3, b3, Wg, bg, Wm1, bm1, Wm2, bm2):
    vmem = pl.BlockSpec(memory_space=pltpu.MemorySpace.VMEM)
    hbm = pl.BlockSpec(memory_space=pltpu.MemorySpace.HBM)
    out = pl.pallas_call(
        _fused,
        out_shape=jax.ShapeDtypeStruct((1, 10, 10, 2), jnp.float32),
        in_specs=[vmem] * 10 + [hbm] * 3,
        out_specs=vmem,
        scratch_shapes=[
            pltpu.VMEM((512, 64), jnp.float32),
            pltpu.VMEM((256, 512), jnp.float32),
            pltpu.VMEM((256, 512), jnp.float32),
            pltpu.VMEM((256, 512), jnp.float32),
            pltpu.VMEM((256, 512), jnp.float32),
            pltpu.VMEM((104, 1024), jnp.float32),
            pltpu.VMEM((96, 1024), jnp.float32),
            pltpu.SemaphoreType.DMA((1 + _WM1_SPLIT + len(_WM2_ROWS),)),
        ],
    )(x, W1, b1, W2, b2, W3, b3, bg, bm1, bm2, Wg, Wm1, Wm2)
    return out


def kernel(x, W1, b1, W2, b2, W3, b3, Wg, bg, Wm1, bm1, Wm2, bm2):
    return _run(x, W1, b1, W2, b2, W3, b3, Wg, bg, Wm1, bm1, Wm2, bm2)


# speedup vs baseline: 1.0756x; 1.0756x over previous
import numpy as np
import jax
import jax.numpy as jnp
from jax.experimental import pallas as pl
from jax.experimental.pallas import tpu as pltpu

_N = 360

_DN_T = (((1,), (1,)), ((), ()))

_WM1_CHUNKS = 1
_WM2_ROWS = (200,)


def _fused(x_ref, w1_ref, b1_ref, w2_ref, b2_ref, w3_ref, b3_ref,
           bg_ref, bm1_ref, bm2_ref, wg_hbm, wm1_hbm, wm2_hbm,
           out_ref, wg_s, wm1_s, wm2_s, sems):
    f32 = jnp.float32

    def mm_t(v, w):
        return jax.lax.dot_general(v, w, _DN_T, preferred_element_type=f32)

    cp_g = pltpu.make_async_copy(wg_hbm, wg_s, sems.at[0])
    cp_g.start()
    cp_m1 = [pltpu.make_async_copy(wm1_hbm, wm1_s, sems.at[1])]
    cp_m1[0].start()
    cp_m2 = [pltpu.make_async_copy(wm2_hbm, wm2_s, sems.at[2])]
    cp_m2[0].start()

    i = jax.lax.broadcasted_iota(jnp.int32, (_N, 1), 0)
    fi = i.astype(f32)

    ang = fi * np.float32(2.0 * np.pi / (_N - 1))
    scan = jnp.transpose(x_ref[0:1, 0:_N])
    nx = scan * jnp.cos(ang)
    ny = scan * jnp.sin(ang)

    end = (i == 0) | (i == (_N - 1))
    dis = jnp.where(end, np.float32(1.0 / np.sqrt(2.0)),
                    np.float32(1.0 / np.sqrt(3.0)))
    cd = dis * dis
    cl = jnp.where(i == 0, 0.0, dis * jnp.roll(dis, 1, axis=0))
    cu = jnp.where(i == (_N - 1), 0.0, dis * jnp.roll(dis, -1, axis=0))

    def agg(v):
        return cd * v + cl * jnp.roll(v, 1, axis=0) + cu * jnp.roll(v, -1, axis=0)

    nodes = jnp.concatenate([nx, ny], axis=1)
    xw = mm_t(nodes, w1_ref[:])
    h = jnp.maximum(agg(xw) + b1_ref[:], 0.0)

    h = jnp.maximum(agg(mm_t(h, w2_ref[:])) + b2_ref[:], 0.0)
    h = jnp.maximum(agg(mm_t(h, w3_ref[:])) + b3_ref[:], 0.0)

    g = jnp.mean(h, axis=0, keepdims=True)
    cp_g.wait()
    c = mm_t(g, wg_s[:]) + bg_ref[:]
    for cp in cp_m1:
        cp.wait()
    m = jnp.maximum(mm_t(c, wm1_s[:]) + bm1_ref[:], 0.0)
    for cp in cp_m2:
        cp.wait()
    row = mm_t(m, wm2_s[:]) + bm2_ref[:]
    out_ref[:] = row.reshape(1, 10, 10, 2)


@jax.jit
def _run(x, W1, b1, W2, b2, W3, b3, Wg, bg, Wm1, bm1, Wm2, bm2):
    vmem = pl.BlockSpec(memory_space=pltpu.MemorySpace.VMEM)
    hbm = pl.BlockSpec(memory_space=pltpu.MemorySpace.HBM)
    out = pl.pallas_call(
        _fused,
        out_shape=jax.ShapeDtypeStruct((1, 10, 10, 2), jnp.float32),
        in_specs=[vmem] * 10 + [hbm] * 3,
        out_specs=vmem,
        scratch_shapes=[
            pltpu.VMEM((512, 64), jnp.float32),
            pltpu.VMEM((1024, 512), jnp.float32),
            pltpu.VMEM((200, 1024), jnp.float32),
            pltpu.SemaphoreType.DMA((1 + _WM1_CHUNKS + len(_WM2_ROWS),)),
        ],
    )(x, W1, b1, W2, b2, W3, b3, bg, bm1, bm2, Wg, Wm1, Wm2)
    return out


def kernel(x, W1, b1, W2, b2, W3, b3, Wg, bg, Wm1, bm1, Wm2, bm2):
    return _run(x, W1, b1, W2, b2, W3, b3, Wg, bg, Wm1, bm1, Wm2, bm2)
